# baseline (speedup 1.0000x reference)
import functools

import jax
import jax.numpy as jnp
from jax import lax
from jax.experimental import pallas as pl
from jax.experimental.pallas import tpu as pltpu
from jax.experimental.pallas import tpu_sc as plsc

STRIDE = 320
HID = 512
CB = 1024
NQ = 4


def _dot_t(a, b):
    return jax.lax.dot_general(a, b, (((1,), (1,)), ((), ())),
                               preferred_element_type=jnp.float32)


def _dot_nt(a, b):
    return jax.lax.dot_general(a, b, (((1,), (0,)), ((), ())),
                               preferred_element_type=jnp.float32)


def _enc_kernel(x_ref, w_ref, b_ref, g_ref, beta_ref, o_ref):
    y = _dot_t(x_ref[...], w_ref[...]) + b_ref[...]
    m = jnp.mean(y, axis=-1, keepdims=True)
    v = jnp.mean((y - m) ** 2, axis=-1, keepdims=True)
    yn = (y - m) * jax.lax.rsqrt(v + 1e-5) * g_ref[...] + beta_ref[...]
    o_ref[...] = jnp.maximum(yn, 0.0)


def _vq_stage_kernel(*refs, nq_prev):
    enc_ref, cb_ref = refs[0], refs[1]
    qrefs = refs[2:2 + nq_prev]
    idx_ref, sn_ref = refs[2 + nq_prev:]
    i = pl.program_id(0)
    res = enc_ref[...]
    for qr in qrefs:
        res = res - qr[...]
    rn = jnp.sum(res * res, axis=-1, keepdims=True)
    cb = cb_ref[0]
    d = rn - 2.0 * _dot_t(res, cb) + jnp.sum(cb * cb, axis=-1)[None, :]
    idx_ref[0, 0, :] = jnp.argmin(d, axis=-1).astype(jnp.int32)
    sv = jnp.sum(rn).reshape(1, 1)

    @pl.when(i == 0)
    def _init():
        sn_ref[...] = sv

    @pl.when(i > 0)
    def _acc():
        sn_ref[...] += sv


def _vq_final_kernel(enc_ref, q0_ref, q1_ref, q2_ref, q3_ref, s1_ref, s2_ref,
                     s3_ref, quant_ref, loss_ref, *, nblk, scale):
    i = pl.program_id(0)
    res = enc_ref[...]
    quant = jnp.zeros_like(res)
    for qr in (q0_ref, q1_ref, q2_ref, q3_ref):
        qv = qr[...]
        res = res - qv
        quant = quant + qv
    quant_ref[...] = quant
    sv = jnp.sum(res * res).reshape(1, 1)

    @pl.when(i == 0)
    def _init():
        loss_ref[...] = sv

    @pl.when(i > 0)
    def _acc():
        loss_ref[...] += sv

    @pl.when(i == nblk - 1)
    def _fin():
        loss_ref[...] = (loss_ref[...] + s1_ref[...] + s2_ref[...]
                         + s3_ref[...]) * scale


def _make_sc_gather(rows_pad, d):
    info = plsc.get_sparse_core_info()
    nw = info.num_cores * info.num_subcores
    b_per_w = rows_pad // nw
    ch = min(128, b_per_w)
    nc = info.num_cores
    mesh = plsc.VectorSubcoreMesh(core_axis_name="c", subcore_axis_name="s")

    @functools.partial(
        pl.kernel, mesh=mesh,
        out_type=jax.ShapeDtypeStruct((rows_pad, d), jnp.float32),
        scratch_types=[
            pltpu.VMEM((ch,), jnp.int32),
            pltpu.VMEM((ch, d), jnp.float32),
            pltpu.SemaphoreType.DMA,
        ],
    )
    def sc_gather(table_hbm, idx_hbm, out_hbm, idx_v, rows_v, sem):
        wid = lax.axis_index("s") * nc + lax.axis_index("c")
        base = wid * b_per_w
        for ci in range(b_per_w // ch):
            off = base + ci * ch
            pltpu.sync_copy(idx_hbm.at[pl.ds(off, ch)], idx_v)
            pltpu.async_copy(table_hbm.at[idx_v], rows_v, sem).wait()
            pltpu.sync_copy(rows_v, out_hbm.at[pl.ds(off, ch)])

    return sc_gather


def _gates_to_hc(gates, c):
    i_g = jax.nn.sigmoid(gates[:, :HID])
    f_g = jax.nn.sigmoid(gates[:, HID:2 * HID])
    g_g = jnp.tanh(gates[:, 2 * HID:3 * HID])
    o_g = jax.nn.sigmoid(gates[:, 3 * HID:])
    c2 = f_g * c + i_g * g_g
    return o_g * jnp.tanh(c2), c2


def _lstm_kernel(x_ref, wih_ref, whh_ref, b_ref, o_ref, xw_ref, h_ref, c_ref,
                 *, bt, bn):
    i = pl.program_id(0)

    @pl.when(i == 0)
    def _init():
        h_ref[...] = jnp.zeros_like(h_ref)
        c_ref[...] = jnp.zeros_like(c_ref)

    xw_ref[...] = _dot_t(x_ref[...], wih_ref[...]) + b_ref[...]

    def step(t, _):
        gates = xw_ref[pl.ds(t * bn, bn), :] + _dot_t(h_ref[...], whh_ref[...])
        h2, c2 = _gates_to_hc(gates, c_ref[...])
        c_ref[...] = c2
        h_ref[...] = h2
        o_ref[pl.ds(t * bn, bn), :] = h2
        return 0

    jax.lax.fori_loop(0, bt, step, 0, unroll=25)


def _lstm_proj_kernel(x_ref, wih_ref, whh_ref, b_ref, outw_ref, outb_ref,
                      o_ref, xw_ref, hbuf_ref, h_ref, c_ref, *, bt, bn):
    i = pl.program_id(0)

    @pl.when(i == 0)
    def _init():
        h_ref[...] = jnp.zeros_like(h_ref)
        c_ref[...] = jnp.zeros_like(c_ref)

    xw_ref[...] = _dot_t(x_ref[...], wih_ref[...]) + b_ref[...]

    def step(t, _):
        gates = xw_ref[pl.ds(t * bn, bn), :] + _dot_t(h_ref[...], whh_ref[...])
        h2, c2 = _gates_to_hc(gates, c_ref[...])
        c_ref[...] = c2
        h_ref[...] = h2
        hbuf_ref[pl.ds(t * bn, bn), :] = h2
        return 0

    jax.lax.fori_loop(0, bt, step, 0, unroll=25)

    o_ref[...] = _dot_t(hbuf_ref[...], outw_ref[...]) + outb_ref[...]


def kernel(waveform, enc_W, enc_b, ln_g, ln_b, codebooks, Wih0, Whh0, bih0,
           bhh0, Wih1, Whh1, bih1, bhh1, out_W, out_b):
    Bn, T = waveform.shape
    frames = T // STRIDE
    rows = Bn * frames
    rb = 1000 if rows % 1000 == 0 else rows

    x = waveform.reshape(rows, STRIDE)

    nblk = rows // rb
    scale = 1.0 / (2.0 * rows * HID)

    enc = pl.pallas_call(
        _enc_kernel,
        grid=(nblk,),
        in_specs=[
            pl.BlockSpec((rb, STRIDE), lambda i: (i, 0)),
            pl.BlockSpec((HID, STRIDE), lambda i: (0, 0)),
            pl.BlockSpec((1, HID), lambda i: (0, 0)),
            pl.BlockSpec((1, HID), lambda i: (0, 0)),
            pl.BlockSpec((1, HID), lambda i: (0, 0)),
        ],
        out_specs=pl.BlockSpec((rb, HID), lambda i: (i, 0)),
        out_shape=jax.ShapeDtypeStruct((rows, HID), jnp.float32),
    )(x, enc_W, enc_b.reshape(1, HID), ln_g.reshape(1, HID),
      ln_b.reshape(1, HID))

    rows_pad = rows + (-rows) % 256
    sc_gather = _make_sc_gather(rows_pad, HID)
    qblock = pl.BlockSpec((rb, HID), lambda i: (i, 0))
    sblock = pl.BlockSpec((1, 1), lambda i: (0, 0))
    qs, snorms = [], []
    for qi in range(NQ):
        idx3, sn = pl.pallas_call(
            functools.partial(_vq_stage_kernel, nq_prev=qi),
            grid=(nblk,),
            in_specs=[qblock,
                      pl.BlockSpec((1, CB, HID),
                                   lambda i, qi=qi: (qi, 0, 0))]
            + [qblock] * qi,
            out_specs=[pl.BlockSpec((1, 1, rb), lambda i: (i, 0, 0)),
                       sblock],
            out_shape=[
                jax.ShapeDtypeStruct((nblk, 1, rb), jnp.int32),
                jax.ShapeDtypeStruct((1, 1), jnp.float32),
            ],
        )(enc, codebooks, *qs)
        snorms.append(sn)
        idx_pad = jnp.concatenate(
            [idx3.reshape(rows), jnp.zeros((rows_pad - rows,), jnp.int32)])
        qs.append(sc_gather(codebooks[qi], idx_pad))

    quant, loss = pl.pallas_call(
        functools.partial(_vq_final_kernel, nblk=nblk, scale=scale),
        grid=(nblk,),
        in_specs=[qblock] + [qblock] * NQ + [sblock] * 3,
        out_specs=[qblock, sblock],
        out_shape=[
            jax.ShapeDtypeStruct((rows, HID), jnp.float32),
            jax.ShapeDtypeStruct((1, 1), jnp.float32),
        ],
    )(enc, *qs, snorms[1], snorms[2], snorms[3])

    dec_in = (quant.reshape(Bn, frames, HID).swapaxes(0, 1)
              .reshape(rows, HID))

    bt_blk = 50 if frames % 50 == 0 else frames
    rbt = bt_blk * Bn
    b0 = (bih0 + bhh0).reshape(1, 4 * HID)
    b1 = (bih1 + bhh1).reshape(1, 4 * HID)

    h0 = pl.pallas_call(
        functools.partial(_lstm_kernel, bt=bt_blk, bn=Bn),
        grid=(frames // bt_blk,),
        in_specs=[
            pl.BlockSpec((rbt, HID), lambda i: (i, 0)),
            pl.BlockSpec((4 * HID, HID), lambda i: (0, 0)),
            pl.BlockSpec((4 * HID, HID), lambda i: (0, 0)),
            pl.BlockSpec((1, 4 * HID), lambda i: (0, 0)),
        ],
        out_specs=pl.BlockSpec((rbt, HID), lambda i: (i, 0)),
        out_shape=jax.ShapeDtypeStruct((rows, HID), jnp.float32),
        scratch_shapes=[
            pltpu.VMEM((rbt, 4 * HID), jnp.float32),
            pltpu.VMEM((Bn, HID), jnp.float32),
            pltpu.VMEM((Bn, HID), jnp.float32),
        ],
    )(dec_in, Wih0, Whh0, b0)

    out_flat = pl.pallas_call(
        functools.partial(_lstm_proj_kernel, bt=bt_blk, bn=Bn),
        grid=(frames // bt_blk,),
        in_specs=[
            pl.BlockSpec((rbt, HID), lambda i: (i, 0)),
            pl.BlockSpec((4 * HID, HID), lambda i: (0, 0)),
            pl.BlockSpec((4 * HID, HID), lambda i: (0, 0)),
            pl.BlockSpec((1, 4 * HID), lambda i: (0, 0)),
            pl.BlockSpec((STRIDE, HID), lambda i: (0, 0)),
            pl.BlockSpec((1, STRIDE), lambda i: (0, 0)),
        ],
        out_specs=pl.BlockSpec((rbt, STRIDE), lambda i: (i, 0)),
        out_shape=jax.ShapeDtypeStruct((rows, STRIDE), jnp.float32),
        scratch_shapes=[
            pltpu.VMEM((rbt, 4 * HID), jnp.float32),
            pltpu.VMEM((rbt, HID), jnp.float32),
            pltpu.VMEM((Bn, HID), jnp.float32),
            pltpu.VMEM((Bn, HID), jnp.float32),
        ],
    )(h0, Wih1, Whh1, b1, out_W, out_b.reshape(1, STRIDE))

    out = (out_flat.reshape(frames, Bn, STRIDE).swapaxes(0, 1)
           .reshape(Bn, frames * STRIDE))
    return out, loss.reshape(())
